# baseline (speedup 1.0000x reference)
import functools

import jax
import jax.numpy as jnp
from jax import lax
from jax.experimental import pallas as pl
from jax.experimental.pallas import tpu as pltpu
from jax.experimental.pallas import tpu_sc as plsc

N = 65536
E = 1048576
B = 16
S = 8
G = 128
NPG = 512
H1 = 256
H2 = 64
SCALAR = 1.0

NC = 2
NS = 16
LANES = 16
NW = NC * NS
GPW = G // NW
EPG = E // G
EPW = EPG * GPW
REP = LANES

def _edge_kernel_body(edge_hbm, out_hbm, src_v, dst_v, hist0, hist1, r0, r1,
                      accv, sem):
    cid = lax.axis_index("c")
    sid = lax.axis_index("s")
    wid = sid * NC + cid
    ebase = wid * EPW
    cps = []
    for j in range(GPW):
        cps.append((
            pltpu.async_copy(
                edge_hbm.at[0, pl.ds(ebase + j * EPG, EPG)],
                src_v.at[pl.ds(j * EPG, EPG)], sem),
            pltpu.async_copy(
                edge_hbm.at[1, pl.ds(ebase + j * EPG, EPG)],
                dst_v.at[pl.ds(j * EPG, EPG)], sem),
        ))

    zeros_i = jnp.zeros((LANES,), jnp.int32)

    for j in range(GPW):
        node_base = (wid * GPW + j) * NPG
        jeb = j * EPG
        cps[j][0].wait()
        cps[j][1].wait()

        @plsc.parallel_loop(0, NPG, LANES, unroll=4)
        def zbody(k):
            hist0[pl.ds(k, LANES)] = zeros_i
            hist1[pl.ds(k, LANES)] = zeros_i

        @plsc.parallel_loop(0, EPG, LANES, unroll=16)
        def hbody(i, jeb=jeb, node_base=node_base):
            off = jeb + i
            sv = src_v[pl.ds(off, LANES)] - node_base
            dv = dst_v[pl.ds(off, LANES)] - node_base
            c0, m0 = plsc.scan_count(sv)
            c1, m1 = plsc.scan_count(dv)
            plsc.addupdate_scatter(hist0, [sv], c0, mask=m0)
            plsc.addupdate_scatter(hist1, [dv], c1, mask=m1)

        @plsc.parallel_loop(0, NPG, LANES, unroll=2)
        def rbody(k):
            for d in range(2):
                hist = hist0 if d == 0 else hist1
                rr = r0 if d == 0 else r1
                tot = hist[pl.ds(k, LANES)]
                x = jnp.maximum(tot.astype(jnp.float32), 1.0)
                yi = 0x5F3759DF - (plsc.bitcast(x, jnp.int32) >> 1)
                y = plsc.bitcast(yi, jnp.float32)
                y = y * (1.5 - 0.5 * x * y * y)
                y = y * (1.5 - 0.5 * x * y * y)
                y = y * (1.5 - 0.5 * x * y * y)
                rr[pl.ds(k, LANES)] = y

        @plsc.parallel_loop(
            0, EPG, LANES, unroll=16,
            carry=jnp.zeros((LANES,), jnp.float32),
        )
        def acc(i, a, jeb=jeb, node_base=node_base):
            off = jeb + i
            sv = src_v[pl.ds(off, LANES)] - node_base
            dv = dst_v[pl.ds(off, LANES)] - node_base
            ga = plsc.load_gather(r0, [sv])
            gb = plsc.load_gather(r1, [dv])
            return a + ga * gb

        accv[pl.ds(j * LANES, LANES)] = acc

    pltpu.sync_copy(accv, out_hbm.at[pl.ds(wid * GPW * LANES, GPW * LANES)])


@functools.lru_cache(maxsize=None)
def _make_edge_kernel():
    mesh = plsc.VectorSubcoreMesh(
        core_axis_name="c", subcore_axis_name="s", num_cores=NC, num_subcores=NS
    )
    return pl.kernel(
        _edge_kernel_body,
        out_type=jax.ShapeDtypeStruct((G * LANES,), jnp.float32),
        mesh=mesh,
        compiler_params=pltpu.CompilerParams(needs_layout_passes=False),
        scratch_types=[
            pltpu.VMEM((EPW,), jnp.int32),
            pltpu.VMEM((EPW,), jnp.int32),
            pltpu.VMEM((NPG,), jnp.int32),
            pltpu.VMEM((NPG,), jnp.int32),
            pltpu.VMEM((NPG,), jnp.float32),
            pltpu.VMEM((NPG,), jnp.float32),
            pltpu.VMEM((GPW * LANES,), jnp.float32),
            pltpu.SemaphoreType.DMA,
        ],
    )


def _head_kernel(accs, scn, W, bconv, wihgT, whhgT, bihg, bhhg,
                 wihlT, whhlT, bihl, bhhl, w1T, b1, w2T, b2, out_ref):
    T = jnp.sum(accs[...], axis=1)
    wsum = jnp.sum(W[...], axis=0, keepdims=True)
    gsum = T[:, None] * wsum + float(NPG) * bconv[...]
    gi_all = (jnp.dot(gsum, wihgT[...], preferred_element_type=jnp.float32)
              + bihg[...]).reshape(B, S, 3 * H2)

    h = jnp.zeros((B, H2), jnp.float32)
    for s in range(S):
        gi = gi_all[:, s, :]
        gh = jnp.dot(h, whhgT[...], preferred_element_type=jnp.float32) + bhhg[...]
        r = jax.nn.sigmoid(gi[:, :H2] + gh[:, :H2])
        z = jax.nn.sigmoid(gi[:, H2:2 * H2] + gh[:, H2:2 * H2])
        n = jnp.tanh(gi[:, 2 * H2:] + r * gh[:, 2 * H2:])
        h = (1.0 - z) * n + z * h

    scnv = scn[...]
    graph_pred = jnp.dot(h, w1T[...], preferred_element_type=jnp.float32) + b1[...]
    rel = jnp.mean(scnv, axis=1, keepdims=True) * (SCALAR / float(NPG))
    graph_pred = jax.nn.sigmoid(graph_pred * rel)

    hl = jnp.zeros((B, H2), jnp.float32)
    cl = jnp.zeros((B, H2), jnp.float32)
    for s in range(S - 1):
        xs = scnv[:, s:s + 1]
        g = (xs * wihlT[...] + bihl[...]
             + jnp.dot(hl, whhlT[...], preferred_element_type=jnp.float32)
             + bhhl[...])
        i = jax.nn.sigmoid(g[:, :H2])
        f = jax.nn.sigmoid(g[:, H2:2 * H2])
        gg = jnp.tanh(g[:, 2 * H2:3 * H2])
        o = jax.nn.sigmoid(g[:, 3 * H2:])
        cl = f * cl + i * gg
        hl = o * jnp.tanh(cl)

    num_pred = jnp.dot(hl, w2T[...], preferred_element_type=jnp.float32) + b2[...]
    pred = 0.2 * graph_pred + 0.8 * num_pred
    target = scnv[:, S - 1:S]
    sq = (pred - target) ** 2
    out_ref[...] = jnp.sum(sq, axis=0, keepdims=True).sum(axis=1, keepdims=True) / float(B)


def kernel(edge_index, node_graph_ids, sc_num, W, b_conv, Wih_g, Whh_g,
           bih_g, bhh_g, Wih_l, Whh_l, bih_l, bhh_l, w1, b1, w2, b2):
    del node_graph_ids
    accs = _make_edge_kernel()(edge_index).reshape(G, LANES)
    loss = pl.pallas_call(
        _head_kernel,
        out_shape=jax.ShapeDtypeStruct((1, 1), jnp.float32),
    )(
        accs, sc_num, W, b_conv.reshape(1, H2),
        Wih_g.T, Whh_g.T, bih_g.reshape(1, 3 * H2), bhh_g.reshape(1, 3 * H2),
        Wih_l.T, Whh_l.T, bih_l.reshape(1, 4 * H2), bhh_l.reshape(1, 4 * H2),
        w1.T, b1.reshape(1, 1), w2.T, b2.reshape(1, 1),
    )
    return loss.reshape(())


# speedup vs baseline: 1.0865x; 1.0865x over previous
import functools

import jax
import jax.numpy as jnp
from jax import lax
from jax.experimental import pallas as pl
from jax.experimental.pallas import tpu as pltpu
from jax.experimental.pallas import tpu_sc as plsc

N = 65536
E = 1048576
B = 16
S = 8
G = 128
NPG = 512
H1 = 256
H2 = 64
SCALAR = 1.0

NC = 2
NS = 16
LANES = 16
NW = NC * NS
GPW = G // NW
EPG = E // G
EPW = EPG * GPW
REP = LANES

def _edge_kernel_body(edge_hbm, out_hbm, src_v, dst_v, hist0, hist1, r0, r1,
                      accv, sem):
    cid = lax.axis_index("c")
    sid = lax.axis_index("s")
    wid = sid * NC + cid
    ebase = wid * EPW
    cps = []
    for j in range(GPW):
        cps.append((
            pltpu.async_copy(
                edge_hbm.at[0, pl.ds(ebase + j * EPG, EPG)],
                src_v.at[pl.ds(j * EPG, EPG)], sem),
            pltpu.async_copy(
                edge_hbm.at[1, pl.ds(ebase + j * EPG, EPG)],
                dst_v.at[pl.ds(j * EPG, EPG)], sem),
        ))

    zeros_i = jnp.zeros((LANES,), jnp.int32)
    del cps

    def jloop(j, _):
        node_base = (wid * GPW + j) * NPG
        jeb = j * EPG
        pltpu.make_async_copy(
            edge_hbm.at[0, pl.ds(0, EPG)], src_v.at[pl.ds(0, EPG)], sem
        ).wait()
        pltpu.make_async_copy(
            edge_hbm.at[0, pl.ds(0, EPG)], src_v.at[pl.ds(0, EPG)], sem
        ).wait()

        @plsc.parallel_loop(0, NPG, LANES, unroll=4)
        def zbody(k):
            hist0[pl.ds(k, LANES)] = zeros_i
            hist1[pl.ds(k, LANES)] = zeros_i

        @plsc.parallel_loop(0, EPG, LANES, unroll=8)
        def hbody(i):
            off = jeb + i
            sv = src_v[pl.ds(off, LANES)] - node_base
            dv = dst_v[pl.ds(off, LANES)] - node_base
            c0, m0 = plsc.scan_count(sv)
            c1, m1 = plsc.scan_count(dv)
            plsc.addupdate_scatter(hist0, [sv], c0, mask=m0)
            plsc.addupdate_scatter(hist1, [dv], c1, mask=m1)

        @plsc.parallel_loop(0, NPG, LANES, unroll=2)
        def rbody(k):
            for d in range(2):
                hist = hist0 if d == 0 else hist1
                rr = r0 if d == 0 else r1
                tot = hist[pl.ds(k, LANES)]
                x = jnp.maximum(tot.astype(jnp.float32), 1.0)
                yi = 0x5F3759DF - (plsc.bitcast(x, jnp.int32) >> 1)
                y = plsc.bitcast(yi, jnp.float32)
                y = y * (1.5 - 0.5 * x * y * y)
                y = y * (1.5 - 0.5 * x * y * y)
                y = y * (1.5 - 0.5 * x * y * y)
                rr[pl.ds(k, LANES)] = y

        @plsc.parallel_loop(
            0, EPG, LANES, unroll=8,
            carry=jnp.zeros((LANES,), jnp.float32),
        )
        def acc(i, a):
            off = jeb + i
            sv = src_v[pl.ds(off, LANES)] - node_base
            dv = dst_v[pl.ds(off, LANES)] - node_base
            ga = plsc.load_gather(r0, [sv])
            gb = plsc.load_gather(r1, [dv])
            return a + ga * gb

        accv[pl.ds(j * LANES, LANES)] = acc
        return 0

    lax.fori_loop(0, GPW, jloop, 0)

    pltpu.sync_copy(accv, out_hbm.at[pl.ds(wid * GPW * LANES, GPW * LANES)])


@functools.lru_cache(maxsize=None)
def _make_edge_kernel():
    mesh = plsc.VectorSubcoreMesh(
        core_axis_name="c", subcore_axis_name="s", num_cores=NC, num_subcores=NS
    )
    return pl.kernel(
        _edge_kernel_body,
        out_type=jax.ShapeDtypeStruct((G * LANES,), jnp.float32),
        mesh=mesh,
        compiler_params=pltpu.CompilerParams(needs_layout_passes=False),
        scratch_types=[
            pltpu.VMEM((EPW,), jnp.int32),
            pltpu.VMEM((EPW,), jnp.int32),
            pltpu.VMEM((NPG,), jnp.int32),
            pltpu.VMEM((NPG,), jnp.int32),
            pltpu.VMEM((NPG,), jnp.float32),
            pltpu.VMEM((NPG,), jnp.float32),
            pltpu.VMEM((GPW * LANES,), jnp.float32),
            pltpu.SemaphoreType.DMA,
        ],
    )


def _head_kernel(accs, scn, W, bconv, wihgT, whhgT, bihg, bhhg,
                 wihlT, whhlT, bihl, bhhl, w1T, b1, w2T, b2, out_ref):
    T = jnp.sum(accs[...], axis=1)
    wsum = jnp.sum(W[...], axis=0, keepdims=True)
    gsum = T[:, None] * wsum + float(NPG) * bconv[...]
    gi_all = (jnp.dot(gsum, wihgT[...], preferred_element_type=jnp.float32)
              + bihg[...]).reshape(B, S, 3 * H2)

    h = jnp.zeros((B, H2), jnp.float32)
    for s in range(S):
        gi = gi_all[:, s, :]
        gh = jnp.dot(h, whhgT[...], preferred_element_type=jnp.float32) + bhhg[...]
        r = jax.nn.sigmoid(gi[:, :H2] + gh[:, :H2])
        z = jax.nn.sigmoid(gi[:, H2:2 * H2] + gh[:, H2:2 * H2])
        n = jnp.tanh(gi[:, 2 * H2:] + r * gh[:, 2 * H2:])
        h = (1.0 - z) * n + z * h

    scnv = scn[...]
    graph_pred = jnp.dot(h, w1T[...], preferred_element_type=jnp.float32) + b1[...]
    rel = jnp.mean(scnv, axis=1, keepdims=True) * (SCALAR / float(NPG))
    graph_pred = jax.nn.sigmoid(graph_pred * rel)

    hl = jnp.zeros((B, H2), jnp.float32)
    cl = jnp.zeros((B, H2), jnp.float32)
    for s in range(S - 1):
        xs = scnv[:, s:s + 1]
        g = (xs * wihlT[...] + bihl[...]
             + jnp.dot(hl, whhlT[...], preferred_element_type=jnp.float32)
             + bhhl[...])
        i = jax.nn.sigmoid(g[:, :H2])
        f = jax.nn.sigmoid(g[:, H2:2 * H2])
        gg = jnp.tanh(g[:, 2 * H2:3 * H2])
        o = jax.nn.sigmoid(g[:, 3 * H2:])
        cl = f * cl + i * gg
        hl = o * jnp.tanh(cl)

    num_pred = jnp.dot(hl, w2T[...], preferred_element_type=jnp.float32) + b2[...]
    pred = 0.2 * graph_pred + 0.8 * num_pred
    target = scnv[:, S - 1:S]
    sq = (pred - target) ** 2
    out_ref[...] = jnp.sum(sq, axis=0, keepdims=True).sum(axis=1, keepdims=True) / float(B)


def kernel(edge_index, node_graph_ids, sc_num, W, b_conv, Wih_g, Whh_g,
           bih_g, bhh_g, Wih_l, Whh_l, bih_l, bhh_l, w1, b1, w2, b2):
    del node_graph_ids
    accs = _make_edge_kernel()(edge_index).reshape(G, LANES)
    loss = pl.pallas_call(
        _head_kernel,
        out_shape=jax.ShapeDtypeStruct((1, 1), jnp.float32),
    )(
        accs, sc_num, W, b_conv.reshape(1, H2),
        Wih_g.T, Whh_g.T, bih_g.reshape(1, 3 * H2), bhh_g.reshape(1, 3 * H2),
        Wih_l.T, Whh_l.T, bih_l.reshape(1, 4 * H2), bhh_l.reshape(1, 4 * H2),
        w1.T, b1.reshape(1, 1), w2.T, b2.reshape(1, 1),
    )
    return loss.reshape(())


# speedup vs baseline: 1.1738x; 1.0803x over previous
import functools

import jax
import jax.numpy as jnp
from jax import lax
from jax.experimental import pallas as pl
from jax.experimental.pallas import tpu as pltpu
from jax.experimental.pallas import tpu_sc as plsc

N = 65536
E = 1048576
B = 16
S = 8
G = 128
NPG = 512
H1 = 256
H2 = 64
SCALAR = 1.0

NC = 2
NS = 16
LANES = 16
NW = NC * NS
GPW = G // NW
EPG = E // G
EPW = EPG * GPW
REP = LANES

def _edge_kernel_body(edge_hbm, out_hbm, src_v, dst_v, hist0, hist1, r0, r1,
                      accv, sem):
    cid = lax.axis_index("c")
    sid = lax.axis_index("s")
    wid = sid * NC + cid
    ebase = wid * EPW
    cps = []
    for j in range(GPW):
        cps.append((
            pltpu.async_copy(
                edge_hbm.at[0, pl.ds(ebase + j * EPG, EPG)],
                src_v.at[pl.ds(j * EPG, EPG)], sem),
            pltpu.async_copy(
                edge_hbm.at[1, pl.ds(ebase + j * EPG, EPG)],
                dst_v.at[pl.ds(j * EPG, EPG)], sem),
        ))

    zeros_i = jnp.zeros((LANES,), jnp.int32)
    del cps

    def jloop(j, _):
        node_base = (wid * GPW + j) * NPG
        jeb = j * EPG
        pltpu.make_async_copy(
            edge_hbm.at[0, pl.ds(0, EPG)], src_v.at[pl.ds(0, EPG)], sem
        ).wait()
        pltpu.make_async_copy(
            edge_hbm.at[0, pl.ds(0, EPG)], src_v.at[pl.ds(0, EPG)], sem
        ).wait()

        @plsc.parallel_loop(0, NPG, LANES, unroll=4)
        def zbody(k):
            hist0[pl.ds(k, LANES)] = zeros_i
            hist1[pl.ds(k, LANES)] = zeros_i

        @plsc.parallel_loop(0, EPG, LANES, unroll=8)
        def hbody(i):
            off = jeb + i
            sv = src_v[pl.ds(off, LANES)] - node_base
            dv = dst_v[pl.ds(off, LANES)] - node_base
            c0, m0 = plsc.scan_count(sv)
            c1, m1 = plsc.scan_count(dv)
            plsc.addupdate_scatter(hist0, [sv], c0, mask=m0)
            plsc.addupdate_scatter(hist1, [dv], c1, mask=m1)

        @plsc.parallel_loop(0, NPG, LANES, unroll=2)
        def rbody(k):
            for d in range(2):
                hist = hist0 if d == 0 else hist1
                rr = r0 if d == 0 else r1
                tot = hist[pl.ds(k, LANES)]
                x = jnp.maximum(tot.astype(jnp.float32), 1.0)
                yi = 0x5F3759DF - (plsc.bitcast(x, jnp.int32) >> 1)
                y = plsc.bitcast(yi, jnp.float32)
                y = y * (1.5 - 0.5 * x * y * y)
                y = y * (1.5 - 0.5 * x * y * y)
                y = y * (1.5 - 0.5 * x * y * y)
                rr[pl.ds(k, LANES)] = y

        @plsc.parallel_loop(
            0, EPG, LANES, unroll=8,
            carry=jnp.zeros((LANES,), jnp.float32),
        )
        def acc(i, a):
            off = jeb + i
            sv = src_v[pl.ds(off, LANES)] - node_base
            dv = dst_v[pl.ds(off, LANES)] - node_base
            ga = plsc.load_gather(r0, [sv])
            gb = plsc.load_gather(r1, [dv])
            return a + ga * gb

        accv[j] = acc
        return 0

    lax.fori_loop(0, GPW, jloop, 0)

    pltpu.sync_copy(accv, out_hbm.at[pl.ds(wid * GPW, GPW)])


@functools.lru_cache(maxsize=None)
def _make_edge_kernel():
    mesh = plsc.VectorSubcoreMesh(
        core_axis_name="c", subcore_axis_name="s", num_cores=NC, num_subcores=NS
    )
    return pl.kernel(
        _edge_kernel_body,
        out_type=jax.ShapeDtypeStruct((G, LANES), jnp.float32),
        mesh=mesh,
        compiler_params=pltpu.CompilerParams(needs_layout_passes=False),
        scratch_types=[
            pltpu.VMEM((EPW,), jnp.int32),
            pltpu.VMEM((EPW,), jnp.int32),
            pltpu.VMEM((NPG,), jnp.int32),
            pltpu.VMEM((NPG,), jnp.int32),
            pltpu.VMEM((NPG,), jnp.float32),
            pltpu.VMEM((NPG,), jnp.float32),
            pltpu.VMEM((GPW, LANES), jnp.float32),
            pltpu.SemaphoreType.DMA,
        ],
    )


def _lstm_kernel(scn, wihlT, whhlT, bihl, bhhl, w2T, b2, out_ref):
    scnv = scn[...]
    hl = jnp.zeros((B, H2), jnp.float32)
    cl = jnp.zeros((B, H2), jnp.float32)
    for s in range(S - 1):
        xs = scnv[:, s:s + 1]
        g = (xs * wihlT[...] + bihl[...]
             + jnp.dot(hl, whhlT[...], preferred_element_type=jnp.float32)
             + bhhl[...])
        i = jax.nn.sigmoid(g[:, :H2])
        f = jax.nn.sigmoid(g[:, H2:2 * H2])
        gg = jnp.tanh(g[:, 2 * H2:3 * H2])
        o = jax.nn.sigmoid(g[:, 3 * H2:])
        cl = f * cl + i * gg
        hl = o * jnp.tanh(cl)
    out_ref[...] = jnp.dot(hl, w2T[...], preferred_element_type=jnp.float32) + b2[...]


def _head_kernel(accs, scn, num_pred_in, W, bconv, wihgT, whhgT, bihg, bhhg,
                 w1T, b1, out_ref):
    T = jnp.sum(accs[...], axis=1)
    wsum = jnp.sum(W[...], axis=0, keepdims=True)
    gsum = T[:, None] * wsum + float(NPG) * bconv[...]
    gi_all = (jnp.dot(gsum, wihgT[...], preferred_element_type=jnp.float32)
              + bihg[...]).reshape(B, S, 3 * H2)

    h = jnp.zeros((B, H2), jnp.float32)
    for s in range(S):
        gi = gi_all[:, s, :]
        gh = jnp.dot(h, whhgT[...], preferred_element_type=jnp.float32) + bhhg[...]
        r = jax.nn.sigmoid(gi[:, :H2] + gh[:, :H2])
        z = jax.nn.sigmoid(gi[:, H2:2 * H2] + gh[:, H2:2 * H2])
        n = jnp.tanh(gi[:, 2 * H2:] + r * gh[:, 2 * H2:])
        h = (1.0 - z) * n + z * h

    scnv = scn[...]
    graph_pred = jnp.dot(h, w1T[...], preferred_element_type=jnp.float32) + b1[...]
    rel = jnp.mean(scnv, axis=1, keepdims=True) * (SCALAR / float(NPG))
    graph_pred = jax.nn.sigmoid(graph_pred * rel)

    pred = 0.2 * graph_pred + 0.8 * num_pred_in[...]
    target = scnv[:, S - 1:S]
    sq = (pred - target) ** 2
    out_ref[...] = jnp.sum(sq, axis=0, keepdims=True).sum(axis=1, keepdims=True) / float(B)


def kernel(edge_index, node_graph_ids, sc_num, W, b_conv, Wih_g, Whh_g,
           bih_g, bhh_g, Wih_l, Whh_l, bih_l, bhh_l, w1, b1, w2, b2):
    del node_graph_ids
    accs = _make_edge_kernel()(edge_index)
    num_pred = pl.pallas_call(
        _lstm_kernel,
        out_shape=jax.ShapeDtypeStruct((B, 1), jnp.float32),
    )(
        sc_num, Wih_l.T, Whh_l.T,
        bih_l.reshape(1, 4 * H2), bhh_l.reshape(1, 4 * H2),
        w2.T, b2.reshape(1, 1),
    )
    loss = pl.pallas_call(
        _head_kernel,
        out_shape=jax.ShapeDtypeStruct((1, 1), jnp.float32),
    )(
        accs, sc_num, num_pred, W, b_conv.reshape(1, H2),
        Wih_g.T, Whh_g.T, bih_g.reshape(1, 3 * H2), bhh_g.reshape(1, 3 * H2),
        w1.T, b1.reshape(1, 1),
    )
    return loss.reshape(())
